# baseline (speedup 1.0000x reference)
import functools

import jax
import jax.numpy as jnp
from jax import lax
from jax.experimental import pallas as pl
from jax.experimental.pallas import tpu as pltpu
from jax.experimental.pallas import tpu_sc as plsc

N_EDGES = 320000
VOCAB = 100000
EMB_DIM = 48
HIDDEN = 128
LANES = 16

NC, NS = 2, 16
NW = NC * NS
ROWS_PER_W = N_EDGES // NW
CHUNK = 80
NCHUNK = ROWS_PER_W // CHUNK
NPAIR = (NCHUNK - 1) // 2


def _gather_sum_body(e0, e1, e2, e3, tbl, s_hbm,
                     i0, i1, i2, i3, rows_a, rows_b, sem_a, sem_b):
    wid = lax.axis_index("s") * NC + lax.axis_index("c")
    base = wid * ROWS_PER_W
    idx_bufs = (i0, i1, i2, i3)

    for f, e_f in enumerate((e0, e1, e2, e3)):
        pltpu.sync_copy(e_f.at[pl.ds(base, ROWS_PER_W)], idx_bufs[f])

    def fire(c, buf, sem):
        off = c * CHUNK
        for f in range(4):
            pltpu.async_copy(tbl.at[idx_bufs[f].at[pl.ds(off, CHUNK)]],
                             buf.at[f], sem)

    def drain(buf, sem):
        for f in range(4):
            pltpu.make_async_copy(tbl.at[idx_bufs[f].at[pl.ds(0, CHUNK)]],
                                  buf.at[f], sem).wait()

    def sum_and_out(c, buf):
        def row_body(r, carry):
            for j in range(EMB_DIM // LANES):
                sl = pl.ds(j * LANES, LANES)
                acc = (buf[0, r, sl] + buf[1, r, sl]
                       + buf[2, r, sl] + buf[3, r, sl])
                buf[0, r, sl] = acc
            return carry

        lax.fori_loop(0, CHUNK, row_body, 0, unroll=2)
        pltpu.sync_copy(buf.at[0, :, pl.ds(0, EMB_DIM)],
                        s_hbm.at[pl.ds(base + c * CHUNK, CHUNK),
                                 pl.ds(0, EMB_DIM)])

    fire(0, rows_a, sem_a)

    def pair_body(cp, carry):
        c0 = cp * 2
        fire(c0 + 1, rows_b, sem_b)
        drain(rows_a, sem_a)
        sum_and_out(c0, rows_a)
        fire(c0 + 2, rows_a, sem_a)
        drain(rows_b, sem_b)
        sum_and_out(c0 + 1, rows_b)
        return carry

    lax.fori_loop(0, NPAIR, pair_body, 0)
    drain(rows_a, sem_a)
    sum_and_out(NCHUNK - 1, rows_a)


_gather_sum_cache = []


def _gather_sum(*args):
    if not _gather_sum_cache:
        _gather_sum_cache.append(functools.partial(
            pl.kernel,
            out_type=jax.ShapeDtypeStruct((N_EDGES, HIDDEN), jnp.float32),
            mesh=plsc.VectorSubcoreMesh(core_axis_name="c",
                                        subcore_axis_name="s",
                                        num_cores=NC, num_subcores=NS),
            scratch_types=[
                pltpu.VMEM((ROWS_PER_W,), jnp.int32),
                pltpu.VMEM((ROWS_PER_W,), jnp.int32),
                pltpu.VMEM((ROWS_PER_W,), jnp.int32),
                pltpu.VMEM((ROWS_PER_W,), jnp.int32),
                pltpu.VMEM((4, CHUNK, HIDDEN), jnp.float32),
                pltpu.VMEM((4, CHUNK, HIDDEN), jnp.float32),
                pltpu.SemaphoreType.DMA,
                pltpu.SemaphoreType.DMA,
            ],
            compiler_params=pltpu.CompilerParams(use_tc_tiling_on_sc=False),
        )(_gather_sum_body))
    return _gather_sum_cache[0](*args)


TC_BLK = 3200


def _proj_body(s_ref, w_ref, b_ref, o_ref):
    h = jnp.dot(s_ref[:, :EMB_DIM], w_ref[...],
                preferred_element_type=jnp.float32) + b_ref[...]
    o_ref[...] = 0.5 * h * (1.0 + lax.erf(h * 0.7071067811865476))


_proj = pl.pallas_call(
    _proj_body,
    grid=(N_EDGES // TC_BLK,),
    in_specs=[
        pl.BlockSpec((TC_BLK, HIDDEN), lambda i: (i, 0)),
        pl.BlockSpec((EMB_DIM, HIDDEN), lambda i: (0, 0)),
        pl.BlockSpec((1, HIDDEN), lambda i: (0, 0)),
    ],
    out_specs=pl.BlockSpec((TC_BLK, HIDDEN), lambda i: (i, 0)),
    out_shape=jax.ShapeDtypeStruct((N_EDGES, HIDDEN), jnp.float32),
)


def kernel(e, emb0, emb1, emb2, emb3, W, b):
    e32 = e.astype(jnp.int32)
    cols = [e32[:, f] + f * VOCAB for f in range(4)]
    pad = ((0, 0), (0, HIDDEN - EMB_DIM))
    tbl = jnp.concatenate([jnp.pad(t, pad)
                           for t in (emb0, emb1, emb2, emb3)], axis=0)
    s = _gather_sum(*cols, tbl)
    return _proj(s, W, b.reshape(1, HIDDEN))


# speedup vs baseline: 1.7126x; 1.7126x over previous
import functools

import jax
import jax.numpy as jnp
from jax import lax
from jax.experimental import pallas as pl
from jax.experimental.pallas import tpu as pltpu
from jax.experimental.pallas import tpu_sc as plsc

N_EDGES = 320000
VOCAB = 100000
EMB_DIM = 48
HIDDEN = 128
LANES = 16

NC, NS = 2, 16
NW = NC * NS
ROWS_PER_W = N_EDGES // NW
CHUNK = 200
NCHUNK = ROWS_PER_W // CHUNK
NPAIR = (NCHUNK - 1) // 2


def _gather_sum_body(e_all, t0, t1, t2, t3, s_hbm,
                     i0, i1, i2, i3, rows_a, rows_b, sem_a, sem_b):
    wid = lax.axis_index("s") * NC + lax.axis_index("c")
    base = wid * ROWS_PER_W
    idx_bufs = (i0, i1, i2, i3)
    tables = (t0, t1, t2, t3)

    for f in range(4):
        pltpu.sync_copy(e_all.at[pl.ds(f * N_EDGES + base, ROWS_PER_W)],
                        idx_bufs[f])

    def fire(c, buf, sem):
        off = c * CHUNK
        for f in range(4):
            pltpu.async_copy(tables[f].at[idx_bufs[f].at[pl.ds(off, CHUNK)]],
                             buf.at[f], sem)

    def drain(buf, sem):
        for f in range(4):
            pltpu.make_async_copy(tables[f].at[idx_bufs[f].at[pl.ds(0, CHUNK)]],
                                  buf.at[f], sem).wait()

    def sum_and_out(c, buf):
        def row_body(r, carry):
            for j in range(EMB_DIM // LANES):
                sl = pl.ds(j * LANES, LANES)
                acc = (buf[0, r, sl] + buf[1, r, sl]
                       + buf[2, r, sl] + buf[3, r, sl])
                buf[0, r, sl] = acc
            return carry

        lax.fori_loop(0, CHUNK, row_body, 0, unroll=2)
        pltpu.sync_copy(buf.at[0],
                        s_hbm.at[pl.ds(base + c * CHUNK, CHUNK),
                                 pl.ds(0, EMB_DIM)])

    fire(0, rows_a, sem_a)

    def pair_body(cp, carry):
        c0 = cp * 2
        fire(c0 + 1, rows_b, sem_b)
        drain(rows_a, sem_a)
        sum_and_out(c0, rows_a)
        fire(c0 + 2, rows_a, sem_a)
        drain(rows_b, sem_b)
        sum_and_out(c0 + 1, rows_b)
        return carry

    lax.fori_loop(0, NPAIR, pair_body, 0)
    fire(NCHUNK - 1, rows_b, sem_b)
    drain(rows_a, sem_a)
    sum_and_out(NCHUNK - 2, rows_a)
    drain(rows_b, sem_b)
    sum_and_out(NCHUNK - 1, rows_b)


_gather_sum_cache = []


def _gather_sum(*args):
    if not _gather_sum_cache:
        _gather_sum_cache.append(functools.partial(
            pl.kernel,
            out_type=jax.ShapeDtypeStruct((N_EDGES, HIDDEN), jnp.float32),
            mesh=plsc.VectorSubcoreMesh(core_axis_name="c",
                                        subcore_axis_name="s",
                                        num_cores=NC, num_subcores=NS),
            scratch_types=[
                pltpu.VMEM((ROWS_PER_W,), jnp.int32),
                pltpu.VMEM((ROWS_PER_W,), jnp.int32),
                pltpu.VMEM((ROWS_PER_W,), jnp.int32),
                pltpu.VMEM((ROWS_PER_W,), jnp.int32),
                pltpu.VMEM((4, CHUNK, EMB_DIM), jnp.float32),
                pltpu.VMEM((4, CHUNK, EMB_DIM), jnp.float32),
                pltpu.SemaphoreType.DMA,
                pltpu.SemaphoreType.DMA,
            ],
            compiler_params=pltpu.CompilerParams(use_tc_tiling_on_sc=False),
        )(_gather_sum_body))
    return _gather_sum_cache[0](*args)


TC_BLK = 3200


def _proj_body(s_ref, w_ref, b_ref, o_ref):
    h = jnp.dot(s_ref[:, :EMB_DIM], w_ref[...],
                preferred_element_type=jnp.float32) + b_ref[...]
    o_ref[...] = 0.5 * h * (1.0 + lax.erf(h * 0.7071067811865476))


_proj = pl.pallas_call(
    _proj_body,
    grid=(N_EDGES // TC_BLK,),
    in_specs=[
        pl.BlockSpec((TC_BLK, HIDDEN), lambda i: (i, 0)),
        pl.BlockSpec((EMB_DIM, HIDDEN), lambda i: (0, 0)),
        pl.BlockSpec((1, HIDDEN), lambda i: (0, 0)),
    ],
    out_specs=pl.BlockSpec((TC_BLK, HIDDEN), lambda i: (i, 0)),
    out_shape=jax.ShapeDtypeStruct((N_EDGES, HIDDEN), jnp.float32),
)


def kernel(e, emb0, emb1, emb2, emb3, W, b):
    e32 = e.astype(jnp.int32)
    e_all = jnp.concatenate([e32[:, f] for f in range(4)])
    s = _gather_sum(e_all, emb0, emb1, emb2, emb3)
    return _proj(s, W, b.reshape(1, HIDDEN))


# speedup vs baseline: 1.7672x; 1.0319x over previous
import functools

import jax
import jax.numpy as jnp
from jax import lax
from jax.experimental import pallas as pl
from jax.experimental.pallas import tpu as pltpu
from jax.experimental.pallas import tpu_sc as plsc

N_EDGES = 320000
VOCAB = 100000
EMB_DIM = 48
HIDDEN = 128
LANES = 16

NC, NS = 2, 16
NW = NC * NS
SEG = 2
ROWS_SEG = N_EDGES // SEG
ROWS_PER_W = ROWS_SEG // NW
CHUNK = 200
NCHUNK = ROWS_PER_W // CHUNK
NPAIR = (NCHUNK - 1) // 2


def _gather_sum_body(seg_base, e_all, t0, t1, t2, t3, s_hbm,
                     i0, i1, i2, i3, rows_a, rows_b, sem_a, sem_b):
    wid = lax.axis_index("s") * NC + lax.axis_index("c")
    base = wid * ROWS_PER_W
    idx_bufs = (i0, i1, i2, i3)
    tables = (t0, t1, t2, t3)

    for f in range(4):
        pltpu.sync_copy(
            e_all.at[pl.ds(f * N_EDGES + seg_base + base, ROWS_PER_W)],
            idx_bufs[f])

    def fire(c, buf, sem):
        off = c * CHUNK
        for f in range(4):
            pltpu.async_copy(tables[f].at[idx_bufs[f].at[pl.ds(off, CHUNK)]],
                             buf.at[f], sem)

    def drain(buf, sem):
        for f in range(4):
            pltpu.make_async_copy(tables[f].at[idx_bufs[f].at[pl.ds(0, CHUNK)]],
                                  buf.at[f], sem).wait()

    def sum_and_out(c, buf):
        def row_body(r, carry):
            for j in range(EMB_DIM // LANES):
                sl = pl.ds(j * LANES, LANES)
                acc = (buf[0, r, sl] + buf[1, r, sl]
                       + buf[2, r, sl] + buf[3, r, sl])
                buf[0, r, sl] = acc
            return carry

        lax.fori_loop(0, CHUNK, row_body, 0, unroll=2)
        pltpu.sync_copy(buf.at[0],
                        s_hbm.at[pl.ds(base + c * CHUNK, CHUNK),
                                 pl.ds(0, EMB_DIM)])

    fire(0, rows_a, sem_a)

    def pair_body(cp, carry):
        c0 = cp * 2
        fire(c0 + 1, rows_b, sem_b)
        drain(rows_a, sem_a)
        sum_and_out(c0, rows_a)
        fire(c0 + 2, rows_a, sem_a)
        drain(rows_b, sem_b)
        sum_and_out(c0 + 1, rows_b)
        return carry

    lax.fori_loop(0, NPAIR, pair_body, 0)
    drain(rows_a, sem_a)
    sum_and_out(NCHUNK - 1, rows_a)


_gather_sum_cache = {}


def _gather_sum(k, *args):
    if k not in _gather_sum_cache:
        _gather_sum_cache[k] = functools.partial(
            pl.kernel,
            out_type=jax.ShapeDtypeStruct((ROWS_SEG, HIDDEN), jnp.float32),
            mesh=plsc.VectorSubcoreMesh(core_axis_name="c",
                                        subcore_axis_name="s",
                                        num_cores=NC, num_subcores=NS),
            scratch_types=[
                pltpu.VMEM((ROWS_PER_W,), jnp.int32),
                pltpu.VMEM((ROWS_PER_W,), jnp.int32),
                pltpu.VMEM((ROWS_PER_W,), jnp.int32),
                pltpu.VMEM((ROWS_PER_W,), jnp.int32),
                pltpu.VMEM((4, CHUNK, EMB_DIM), jnp.float32),
                pltpu.VMEM((4, CHUNK, EMB_DIM), jnp.float32),
                pltpu.SemaphoreType.DMA,
                pltpu.SemaphoreType.DMA,
            ],
            compiler_params=pltpu.CompilerParams(use_tc_tiling_on_sc=False),
        )(functools.partial(_gather_sum_body, k * ROWS_SEG))
    return _gather_sum_cache[k](*args)


TC_BLK = 3200
SEG_BLKS = ROWS_SEG // TC_BLK


def _proj_body(s_ref, w_ref, b_ref, o_ref):
    h = jnp.dot(s_ref[:, :EMB_DIM], w_ref[...],
                preferred_element_type=jnp.float32) + b_ref[...]
    o_ref[...] = 0.5 * h * (1.0 + lax.erf(h * 0.7071067811865476))


def _make_proj(k, aliased):
    in_specs = [
        pl.BlockSpec((TC_BLK, HIDDEN), lambda i: (i, 0)),
        pl.BlockSpec((EMB_DIM, HIDDEN), lambda i: (0, 0)),
        pl.BlockSpec((1, HIDDEN), lambda i: (0, 0)),
    ]
    kwargs = {}
    if aliased:
        in_specs.append(pl.BlockSpec(memory_space=pl.ANY))
        kwargs["input_output_aliases"] = {3: 0}
    return pl.pallas_call(
        _proj_body,
        grid=(SEG_BLKS,),
        in_specs=in_specs,
        out_specs=pl.BlockSpec((TC_BLK, HIDDEN),
                               lambda i, _k=k: (i + _k * SEG_BLKS, 0)),
        out_shape=jax.ShapeDtypeStruct((N_EDGES, HIDDEN), jnp.float32),
        **kwargs,
    )


def _proj_body_aliased(s_ref, w_ref, b_ref, prev_ref, o_ref):
    _proj_body(s_ref, w_ref, b_ref, o_ref)


def kernel(e, emb0, emb1, emb2, emb3, W, b):
    e32 = e.astype(jnp.int32)
    e_all = jnp.concatenate([e32[:, f] for f in range(4)])
    b2 = b.reshape(1, HIDDEN)
    tabs = (emb0, emb1, emb2, emb3)

    s0 = _gather_sum(0, e_all, *tabs)
    s1 = _gather_sum(1, e_all, *tabs)
    out = _make_proj(0, aliased=False)(s0, W, b2)
    out = pl.pallas_call(
        _proj_body_aliased,
        grid=(SEG_BLKS,),
        in_specs=[
            pl.BlockSpec((TC_BLK, HIDDEN), lambda i: (i, 0)),
            pl.BlockSpec((EMB_DIM, HIDDEN), lambda i: (0, 0)),
            pl.BlockSpec((1, HIDDEN), lambda i: (0, 0)),
            pl.BlockSpec(memory_space=pl.ANY),
        ],
        out_specs=pl.BlockSpec((TC_BLK, HIDDEN),
                               lambda i: (i + SEG_BLKS, 0)),
        out_shape=jax.ShapeDtypeStruct((N_EDGES, HIDDEN), jnp.float32),
        input_output_aliases={3: 0},
    )(s1, W, b2, out)
    return out
